# baseline (speedup 1.0000x reference)
import functools

import jax
import jax.numpy as jnp
from jax import lax
from jax.experimental import pallas as pl
from jax.experimental.pallas import tpu as pltpu

B, C, H, W = 8, 384, 224, 224
HID, E, K = 96, 64, 2
S = H * W
CHUNK = 1024
NSTEPS = S // CHUNK


def _fused_body(x_ref, w1_ref, b1_ref, w2_ref, b2_ref,
                idx_ref, val_ref, probs_ref, acc_ref):
    j = pl.program_id(0)

    @pl.when(j == 0)
    def _init():
        acc_ref[...] = jnp.zeros_like(acc_ref)

    acc_ref[...] += jnp.sum(x_ref[...], axis=2)

    @pl.when(j == NSTEPS - 1)
    def _router():
        h = acc_ref[...] * (1.0 / S)
        hid = jnp.dot(h, w1_ref[...], preferred_element_type=jnp.float32)
        hid = jnp.maximum(hid + b1_ref[...], 0.0)
        logits = jnp.dot(hid, w2_ref[...], preferred_element_type=jnp.float32)
        logits = logits + b2_ref[...]
        m = jnp.max(logits, axis=1, keepdims=True)
        e = jnp.exp(logits - m)
        p = e / jnp.sum(e, axis=1, keepdims=True)
        probs_ref[...] = p
        iota = lax.broadcasted_iota(jnp.int32, p.shape, 1)
        m1 = jnp.max(p, axis=1, keepdims=True)
        i1 = jnp.min(jnp.where(p == m1, iota, E), axis=1, keepdims=True)
        p2 = jnp.where(iota == i1, -jnp.inf, p)
        m2 = jnp.max(p2, axis=1, keepdims=True)
        i2 = jnp.min(jnp.where(p2 == m2, iota, E), axis=1, keepdims=True)
        val_ref[...] = jnp.concatenate([m1, m2], axis=1)
        idx_ref[...] = jnp.concatenate([i1, i2], axis=1)


@jax.jit
def kernel(x, W1, b1, W2, b2):
    xr = x.reshape(B, C, S)
    w1t = W1.T
    w2t = W2.T
    b1r = b1.reshape(1, HID)
    b2r = b2.reshape(1, E)

    grid = (NSTEPS,)
    out = pl.pallas_call(
        _fused_body,
        grid=grid,
        in_specs=[
            pl.BlockSpec((B, C, CHUNK), lambda j: (0, 0, j)),
            pl.BlockSpec((C, HID), lambda j: (0, 0)),
            pl.BlockSpec((1, HID), lambda j: (0, 0)),
            pl.BlockSpec((HID, E), lambda j: (0, 0)),
            pl.BlockSpec((1, E), lambda j: (0, 0)),
        ],
        out_specs=[
            pl.BlockSpec((B, K), lambda j: (0, 0)),
            pl.BlockSpec((B, K), lambda j: (0, 0)),
            pl.BlockSpec((B, E), lambda j: (0, 0)),
        ],
        out_shape=[
            jax.ShapeDtypeStruct((B, K), jnp.int32),
            jax.ShapeDtypeStruct((B, K), jnp.float32),
            jax.ShapeDtypeStruct((B, E), jnp.float32),
        ],
        scratch_shapes=[pltpu.VMEM((B, C), jnp.float32)],
        compiler_params=pltpu.CompilerParams(
            dimension_semantics=("arbitrary",),
        ),
    )(xr, w1t, b1r, w2t, b2r)
    topk_idx, topk_val, probs = out
    return (topk_idx, topk_val, probs)
